# baseline (speedup 1.0000x reference)
import functools

import jax
import jax.numpy as jnp
from jax import lax
from jax.experimental import pallas as pl
from jax.experimental.pallas import tpu as pltpu
from jax.experimental.pallas import tpu_sc as plsc

NUM_CLASSES = 100000
FEAT_DIM = 64
BATCH = 1024
ALPHA = 0.5
BETA = 0.05
MARGIN = 15.0

NC, NS = 2, 16
NW = NC * NS
B_PER_W = BATCH // NW


def _worker_id():
    return lax.axis_index("s") * NC + lax.axis_index("c")


@functools.cache
def _sc_scatter_kernel():
    @functools.partial(
        pl.kernel,
        out_type=(),
        mesh=plsc.VectorSubcoreMesh(core_axis_name="c", subcore_axis_name="s",
                                    num_cores=NC, num_subcores=NS),
        scratch_types=[
            pltpu.VMEM((B_PER_W,), jnp.int32),
            pltpu.VMEM((B_PER_W,), jnp.int32),
            pltpu.VMEM((B_PER_W, FEAT_DIM), jnp.float32),
            pltpu.SemaphoreType.DMA,
            pltpu.SemaphoreType.DMA,
        ],
    )
    def _sc_scatter(rows_hbm, win_hbm, lab_hbm, table_ref,
                    win_s, lab_s, rows_v, sem1, sem2):
        base = _worker_id() * B_PER_W
        pltpu.sync_copy(win_hbm.at[pl.ds(base, B_PER_W)], win_s)
        pltpu.sync_copy(lab_hbm.at[pl.ds(base, B_PER_W)], lab_s)
        handles = []
        for g in range(B_PER_W // 16):
            vec = win_s[pl.ds(g * 16, 16)]
            for l in range(16):
                handles.append(pltpu.async_copy(
                    rows_hbm.at[pl.ds(vec[l], 1)],
                    rows_v.at[pl.ds(g * 16 + l, 1)], sem1))
        for h in handles:
            h.wait()
        handles = []
        for g in range(B_PER_W // 16):
            vec = lab_s[pl.ds(g * 16, 16)]
            for l in range(16):
                handles.append(pltpu.async_copy(
                    rows_v.at[pl.ds(g * 16 + l, 1)],
                    table_ref.at[pl.ds(vec[l], 1)], sem2))
        for h in handles:
            h.wait()

    return _sc_scatter


def _dense_body(lab_smem, feat_ref, labc_ref, labr_ref, table_any,
                rows_ref, win_ref, loss_ref, cent_v, sem):
    def issue(i, _):
        r = lab_smem[i]
        pltpu.make_async_copy(
            table_any.at[pl.ds(r, 1)], cent_v.at[pl.ds(i, 1)], sem).start()
        return 0
    lax.fori_loop(0, BATCH, issue, 0, unroll=8)

    def drain(i, _):
        pltpu.make_async_copy(
            table_any.at[pl.ds(0, 1)], cent_v.at[pl.ds(0, 1)], sem).wait()
        return 0
    lax.fori_loop(0, BATCH, drain, 0, unroll=8)

    c = cent_v[...]
    f = feat_ref[...]
    labc = labc_ref[...]
    labr = labr_ref[...]

    sq_col = jnp.sum(c * c, axis=1, keepdims=True)
    sq_row = sq_col.reshape(1, BATCH)
    g = lax.dot_general(c, c, (((1,), (1,)), ((), ())),
                        preferred_element_type=jnp.float32,
                        precision=lax.Precision.HIGHEST)
    d2 = jnp.maximum(sq_col + sq_row - 2.0 * g, 0.0)
    dist = jnp.sqrt(d2)

    neq = (labc != labr)
    mask = jnp.where(neq & (dist <= MARGIN), 1.0, 0.0)

    nd = -dist
    min_v = jnp.min(nd * mask, axis=1, keepdims=True)
    numer = jnp.exp(nd - min_v) * mask
    numer = jnp.where(mask == 0.0, 0.0, numer)
    z = jnp.sum(numer, axis=1, keepdims=True) + 1e-06
    w = numer / z

    s = jnp.sum(w, axis=1, keepdims=True)
    wc = lax.dot_general(w, c, (((1,), (0,)), ((), ())),
                         preferred_element_type=jnp.float32,
                         precision=lax.Precision.HIGHEST)
    delta2 = c * s - wc
    delta2 = jnp.where(jnp.sum(mask) < 1.0, 0.0, delta2)

    rows_ref[...] = c - ALPHA * (c - f) - BETA * delta2

    jiota = lax.broadcasted_iota(jnp.int32, (BATCH, BATCH), 1)
    win_ref[...] = jnp.max(jnp.where(labc == labr, jiota, -1),
                           axis=1, keepdims=True)

    diff = c - f
    loss = jnp.mean(jnp.clip(diff * diff, 1e-12, 1e12))
    loss_ref[...] = jnp.broadcast_to(loss, (1, 1))


_dense = pl.pallas_call(
    _dense_body,
    in_specs=[
        pl.BlockSpec(memory_space=pltpu.SMEM),
        pl.BlockSpec(memory_space=pltpu.VMEM),
        pl.BlockSpec(memory_space=pltpu.VMEM),
        pl.BlockSpec(memory_space=pltpu.VMEM),
        pl.BlockSpec(memory_space=pltpu.HBM),
    ],
    out_shape=(
        jax.ShapeDtypeStruct((BATCH, FEAT_DIM), jnp.float32),
        jax.ShapeDtypeStruct((BATCH, 1), jnp.int32),
        jax.ShapeDtypeStruct((1, 1), jnp.float32),
    ),
    scratch_shapes=[
        pltpu.VMEM((BATCH, FEAT_DIM), jnp.float32),
        pltpu.SemaphoreType.DMA,
    ],
    compiler_params=pltpu.CompilerParams(
        vmem_limit_bytes=100 * 1024 * 1024),
)


def kernel(features, labels, centers_table):
    labels = labels.astype(jnp.int32)
    rows, winner, loss = _dense(
        labels, features,
        labels.reshape(BATCH, 1), labels.reshape(1, BATCH),
        centers_table)
    table_ref = jax.new_ref(centers_table)
    _sc_scatter_kernel()(rows, winner.reshape(BATCH), labels, table_ref)
    new_table = table_ref[...]
    return loss[0, 0], new_table
